# baseline (speedup 1.0000x reference)
import functools

import jax
import jax.numpy as jnp
from jax import lax
from jax.experimental import pallas as pl
from jax.experimental.pallas import tpu as pltpu
from jax.experimental.pallas import tpu_sc as plsc

NC = 2
NS = 16
NW = NC * NS

NB = 4096
S = 50
SP = 56
D = 64
DP = 128
BPC = 2
CW = BPC * SP
N_CHUNKS = NB // NW // BPC
NBUF = 8
K = 4

_mesh = plsc.VectorSubcoreMesh(
    core_axis_name="c", subcore_axis_name="s", num_cores=NC, num_subcores=NS
)


@functools.partial(
    pl.kernel,
    out_type=jax.ShapeDtypeStruct((NB // BPC, CW, DP), jnp.float32),
    mesh=_mesh,
    scratch_types=[
        pltpu.VMEM((N_CHUNKS, DP), jnp.int32),
        pltpu.VMEM((NBUF, CW, DP), jnp.float32),
        [pltpu.SemaphoreType.DMA] * NBUF,
        [pltpu.SemaphoreType.DMA] * NBUF,
    ],
)
def _emb_kernel(table_hbm, idx_hbm, out_hbm, idx_v, rows_v, gsem, wsem):
    wid = lax.axis_index("s") * NC + lax.axis_index("c")
    pltpu.sync_copy(idx_hbm.at[wid], idx_v)
    c0 = wid * N_CHUNKS

    def start_gather(g, b):
        pltpu.async_copy(
            table_hbm.at[idx_v.at[g].at[pl.ds(0, CW)]], rows_v.at[b], gsem[b]
        )

    def wait_gather(g, b):
        pltpu.make_async_copy(
            table_hbm.at[idx_v.at[g].at[pl.ds(0, CW)]], rows_v.at[b], gsem[b]
        ).wait()

    def start_write(g, b):
        pltpu.async_copy(rows_v.at[b], out_hbm.at[c0 + g], wsem[b])

    def wait_write(g, b):
        pltpu.make_async_copy(
            rows_v.at[b], out_hbm.at[c0 + g], wsem[b]
        ).wait()

    for b in range(K):
        start_gather(b, b)
    for b in range(NBUF):
        g = b
        h = g + K
        bh = h % NBUF
        wait_gather(g, b)
        start_write(g, b)
        if h < NBUF:
            start_gather(h, bh)
        else:
            wait_write(h - NBUF, bh)
            start_gather(h, bh)

    def round_body(t, carry):
        for b in range(NBUF):
            g = t * NBUF + b
            h = g + K
            bh = (b + K) % NBUF
            wait_gather(g, b)
            start_write(g, b)
            wait_write(h - NBUF, bh)
            start_gather(h, bh)
        return carry

    lax.fori_loop(1, N_CHUNKS // NBUF - 1, round_body, 0)

    t_last = N_CHUNKS // NBUF - 1
    for b in range(NBUF):
        g = t_last * NBUF + b
        h = g + K
        bh = (b + K) % NBUF
        wait_gather(g, b)
        start_write(g, b)
        if h < N_CHUNKS:
            wait_write(h - NBUF, bh)
            start_gather(h, bh)

    for b in range(NBUF):
        g = t_last * NBUF + b
        wait_write(g, b)


def kernel(x, table):
    xi = x.astype(jnp.int32)
    xp = jnp.concatenate([xi, xi[:, : SP - S]], axis=1)
    idx = xp.reshape(NB // BPC, CW)
    idx = jnp.pad(idx, ((0, 0), (0, DP - CW)))
    idx = idx.reshape(NW, N_CHUNKS, DP)
    table_p = jnp.pad(table, ((0, 0), (0, DP - D)))
    out = _emb_kernel(table_p, idx)
    return out.reshape(NB, SP, DP)[:, :S, :D]


# speedup vs baseline: 1.0043x; 1.0043x over previous
import functools

import jax
import jax.numpy as jnp
from jax import lax
from jax.experimental import pallas as pl
from jax.experimental.pallas import tpu as pltpu
from jax.experimental.pallas import tpu_sc as plsc

NC = 2
NS = 16
NW = NC * NS

NB = 4096
S = 50
SP = 56
D = 64
DP = 128
BPC = 2
CW = BPC * SP
N_CHUNKS = NB // NW // BPC
NBUF = 8
K = 6

_mesh = plsc.VectorSubcoreMesh(
    core_axis_name="c", subcore_axis_name="s", num_cores=NC, num_subcores=NS
)


@functools.partial(
    pl.kernel,
    out_type=jax.ShapeDtypeStruct((NB // BPC, CW, DP), jnp.float32),
    mesh=_mesh,
    scratch_types=[
        pltpu.VMEM((N_CHUNKS, DP), jnp.int32),
        pltpu.VMEM((NBUF, CW, DP), jnp.float32),
        [pltpu.SemaphoreType.DMA] * NBUF,
        [pltpu.SemaphoreType.DMA] * NBUF,
    ],
)
def _emb_kernel(table_hbm, idx_hbm, out_hbm, idx_v, rows_v, gsem, wsem):
    wid = lax.axis_index("s") * NC + lax.axis_index("c")
    pltpu.sync_copy(idx_hbm.at[wid], idx_v)
    c0 = wid * N_CHUNKS

    def start_gather(g, b):
        pltpu.async_copy(
            table_hbm.at[idx_v.at[g].at[pl.ds(0, CW)]], rows_v.at[b], gsem[b]
        )

    def wait_gather(g, b):
        pltpu.make_async_copy(
            table_hbm.at[idx_v.at[g].at[pl.ds(0, CW)]], rows_v.at[b], gsem[b]
        ).wait()

    def start_write(g, b):
        pltpu.async_copy(rows_v.at[b], out_hbm.at[c0 + g], wsem[b])

    def wait_write(g, b):
        pltpu.make_async_copy(
            rows_v.at[b], out_hbm.at[c0 + g], wsem[b]
        ).wait()

    for b in range(K):
        start_gather(b, b)
    for b in range(NBUF):
        g = b
        h = g + K
        bh = h % NBUF
        wait_gather(g, b)
        start_write(g, b)
        if h < NBUF:
            start_gather(h, bh)
        else:
            wait_write(h - NBUF, bh)
            start_gather(h, bh)

    def round_body(t, carry):
        for b in range(NBUF):
            g = t * NBUF + b
            h = g + K
            bh = (b + K) % NBUF
            wait_gather(g, b)
            start_write(g, b)
            wait_write(h - NBUF, bh)
            start_gather(h, bh)
        return carry

    lax.fori_loop(1, N_CHUNKS // NBUF - 1, round_body, 0)

    t_last = N_CHUNKS // NBUF - 1
    for b in range(NBUF):
        g = t_last * NBUF + b
        h = g + K
        bh = (b + K) % NBUF
        wait_gather(g, b)
        start_write(g, b)
        if h < N_CHUNKS:
            wait_write(h - NBUF, bh)
            start_gather(h, bh)

    for b in range(NBUF):
        g = t_last * NBUF + b
        wait_write(g, b)


def kernel(x, table):
    xi = x.astype(jnp.int32)
    xp = jnp.concatenate([xi, xi[:, : SP - S]], axis=1)
    idx = xp.reshape(NB // BPC, CW)
    idx = jnp.pad(idx, ((0, 0), (0, DP - CW)))
    idx = idx.reshape(NW, N_CHUNKS, DP)
    table_p = jnp.pad(table, ((0, 0), (0, DP - D)))
    out = _emb_kernel(table_p, idx)
    return out.reshape(NB, SP, DP)[:, :S, :D]


# speedup vs baseline: 1.0045x; 1.0002x over previous
import functools

import jax
import jax.numpy as jnp
from jax import lax
from jax.experimental import pallas as pl
from jax.experimental.pallas import tpu as pltpu
from jax.experimental.pallas import tpu_sc as plsc

NC = 2
NS = 16
NW = NC * NS

NB = 4096
S = 50
SP = 56
D = 64
DP = 128
BPC = 2
CW = BPC * SP
N_CHUNKS = NB // NW // BPC
NBUF = 8
K = 6

_mesh = plsc.VectorSubcoreMesh(
    core_axis_name="c", subcore_axis_name="s", num_cores=NC, num_subcores=NS
)


@functools.partial(
    pl.kernel,
    out_type=jax.ShapeDtypeStruct((NB // BPC, CW, DP), jnp.float32),
    mesh=_mesh,
    scratch_types=[
        pltpu.VMEM((N_CHUNKS, DP), jnp.int32),
        pltpu.VMEM((NBUF, CW, DP), jnp.float32),
        [pltpu.SemaphoreType.DMA] * NBUF,
        [pltpu.SemaphoreType.DMA] * NBUF,
    ],
)
def _emb_kernel(table_hbm, idx_hbm, out_hbm, idx_v, rows_v, gsem, wsem):
    wid = lax.axis_index("s") * NC + lax.axis_index("c")
    pltpu.sync_copy(idx_hbm.at[wid], idx_v)
    c0 = wid * N_CHUNKS

    def start_gather(g, b):
        pltpu.async_copy(
            table_hbm.at[idx_v.at[g].at[pl.ds(0, CW)]], rows_v.at[b], gsem[b]
        )

    def wait_gather(g, b):
        pltpu.make_async_copy(
            table_hbm.at[idx_v.at[g].at[pl.ds(0, CW)]], rows_v.at[b], gsem[b]
        ).wait()

    def start_write(g, b):
        pltpu.async_copy(rows_v.at[b], out_hbm.at[c0 + g], wsem[b])

    def wait_write(g, b):
        pltpu.make_async_copy(
            rows_v.at[b], out_hbm.at[c0 + g], wsem[b]
        ).wait()

    for b in range(K):
        start_gather(b, b)
    for b in range(NBUF):
        g = b
        h = g + K
        bh = h % NBUF
        wait_gather(g, b)
        start_write(g, b)
        if h < NBUF:
            start_gather(h, bh)
        else:
            wait_write(h - NBUF, bh)
            start_gather(h, bh)

    def round_body(t, carry):
        for b in range(NBUF):
            g = t * NBUF + b
            h = g + K
            bh = (b + K) % NBUF
            wait_gather(g, b)
            start_write(g, b)
            wait_write(h - NBUF, bh)
            start_gather(h, bh)
        return carry

    lax.fori_loop(1, N_CHUNKS // NBUF - 1, round_body, 0)

    t_last = N_CHUNKS // NBUF - 1
    for b in range(NBUF):
        g = t_last * NBUF + b
        h = g + K
        bh = (b + K) % NBUF
        wait_gather(g, b)
        start_write(g, b)
        if h < N_CHUNKS:
            wait_write(h - NBUF, bh)
            start_gather(h, bh)

    for b in range(NBUF):
        g = t_last * NBUF + b
        wait_write(g, b)


def kernel(x, table):
    xi = x.astype(jnp.int32)
    xp = jnp.concatenate([xi, xi[:, : SP - S]], axis=1)
    idx = xp.reshape(NB // BPC, CW)
    idx = jnp.pad(idx, ((0, 0), (0, DP - CW)))
    idx = idx.reshape(NW, N_CHUNKS, DP)
    idx = jnp.where(idx >= 0, idx, jnp.int32(0))
    table_p = jnp.pad(table, ((0, 0), (0, DP - D)))
    out = _emb_kernel(table_p, idx)
    return out.reshape(NB, SP, DP)[:, :S, :D]
